# baseline (speedup 1.0000x reference)
import jax
import jax.numpy as jnp
from jax.experimental import pallas as pl
from jax.experimental.pallas import tpu as pltpu

B = 8
N1 = 4096
N2 = 4096
K = 32
QT = 512
NG = 8
NJ = 64
NS = 8
NR = NG * NJ
R = 6
BIGF = float(2 ** 24)
INF = float("inf")


def _repmin(x):
    x = jnp.minimum(x, pltpu.roll(x, 4, 0))
    x = jnp.minimum(x, pltpu.roll(x, 2, 0))
    x = jnp.minimum(x, pltpu.roll(x, 1, 0))
    return x


def _knn_kernel(x2t_ref, x1_ref, xx_ref, rio_ref, out_ref):
    x2 = x2t_ref[0]
    q = x1_ref[0]
    yy = (q[0:1] * q[0:1] + q[1:2] * q[1:2]) + q[2:3] * q[2:3]
    xxb = jnp.broadcast_to(xx_ref[0], (N2, QT))
    cr = jax.lax.dot_general(
        x2, -2.0 * q, (((1,), (0,)), ((), ())),
        preferred_element_type=jnp.float32)
    dist = jnp.maximum((xxb + yy) + cr, 0.0)
    dist3 = dist.reshape(NR, NS, QT)
    rio = rio_ref[...].reshape(NR, NS, QT)

    vg = [dist3[g * NJ:(g + 1) * NJ] for g in range(NG)]
    rg = [rio[g * NJ:(g + 1) * NJ] for g in range(NG)]
    pvals, pidxs = [], []
    for r in range(R):
        for g in range(NG):
            v = vg[g]
            bmin = jnp.min(v, axis=0)
            bidx = jnp.min(jnp.where(v == bmin[None], rg[g], BIGF),
                           axis=0)
            pvals.append(bmin)
            pidxs.append(bidx)
            vg[g] = jnp.where(rg[g] == bidx[None], INF, v)

    pv = jnp.stack(pvals, axis=0)
    pi = jnp.stack(pidxs, axis=0)
    m = None
    for k in range(K):
        m = _repmin(jnp.min(pv, axis=0))
        cand = jnp.where(pv == m[None], pi, BIGF)
        idx = _repmin(jnp.min(cand, axis=0))
        out_ref[0, pl.ds(k, 1)] = idx[0:1].astype(jnp.int32)
        if k < K - 1:
            pv = jnp.where(pi == idx[None], INF, pv)
    m32 = m

    rem = vg[0]
    for g in range(1, NG):
        rem = jnp.minimum(rem, vg[g])
    rem = _repmin(jnp.min(rem, axis=0))
    bad = jnp.any(rem <= m32)

    @pl.when(bad)
    def _():
        def body(k, vals):
            fm = _repmin(jnp.min(vals, axis=0))
            cand = jnp.where(vals == fm[None], rio, BIGF)
            idx = _repmin(jnp.min(cand, axis=0))
            out_ref[0, pl.ds(k, 1)] = idx[0:1].astype(jnp.int32)
            return jnp.where(rio == idx[None], INF, vals)

        jax.lax.fori_loop(0, K, body, dist3)


def _knn_call(x2t, x1, xx2, rio):
    bsh = x2t.shape[0]
    return pl.pallas_call(
        _knn_kernel,
        grid=(bsh, N1 // QT),
        in_specs=[
            pl.BlockSpec((1, N2, 3), lambda b, i: (b, 0, 0)),
            pl.BlockSpec((1, 3, QT), lambda b, i: (b, 0, i)),
            pl.BlockSpec((1, N2, 1), lambda b, i: (b, 0, 0)),
            pl.BlockSpec((N2, QT), lambda b, i: (0, 0)),
        ],
        out_specs=pl.BlockSpec((1, K, QT), lambda b, i: (b, 0, i)),
        out_shape=jax.ShapeDtypeStruct((bsh, K, N1), jnp.int32),
        compiler_params=pltpu.CompilerParams(
            dimension_semantics=("parallel", "parallel")),
    )(x2t, x1, xx2, rio)


def _shard_fn(xyz2_sh, xyz1_sh):
    x2t = jnp.transpose(xyz2_sh, (0, 2, 1))
    xx2 = jnp.sum(xyz2_sh ** 2, axis=1)[..., None]
    rio = jnp.broadcast_to(
        jnp.arange(N2, dtype=jnp.float32)[:, None], (N2, QT))
    out = _knn_call(x2t, xyz1_sh, xx2, rio)
    return jnp.transpose(out, (0, 2, 1))


@jax.jit
def kernel(xyz2, xyz1):
    ndev = len(jax.devices())
    nsh = 2 if (ndev >= 2 and B % 2 == 0) else 1
    if nsh > 1:
        mesh = jax.make_mesh((nsh,), ("d",))
        p = jax.sharding.PartitionSpec
        nsrep = jax.sharding.NamedSharding(mesh, p())
        bsh = B // nsh

        def _sliced(xyz2_rep, xyz1_rep):
            st = jax.lax.axis_index("d") * bsh
            return _shard_fn(
                jax.lax.dynamic_slice_in_dim(xyz2_rep, st, bsh, 0),
                jax.lax.dynamic_slice_in_dim(xyz1_rep, st, bsh, 0))

        f = jax.shard_map(
            _sliced, mesh=mesh,
            in_specs=(p(), p()),
            out_specs=p("d"), check_vma=False)
        return f(jax.reshard(xyz2, nsrep), jax.reshard(xyz1, nsrep))
    return _shard_fn(xyz2, xyz1)
